# baseline (speedup 1.0000x reference)
import jax
import jax.numpy as jnp
from jax import lax
from jax.experimental import pallas as pl
from jax.experimental.pallas import tpu as pltpu
from jax.experimental.pallas import tpu_sc as plsc

_B, _L, _D = 4096, 200, 64
_N = _B * _L
_NC, _NS = 2, 16
_NW = _NC * _NS
_K = 4
_C = _K * 128
_CHUNKS = _N // (_NW * _C)


def _sc_body(idx_hbm, emb_hbm, tab_hbm, out_hbm, idx_v, pe_v, esem, gsem):
    wid = lax.axis_index("s") * _NC + lax.axis_index("c")

    def chunk(i, carry):
        r0 = (wid * _CHUNKS + i) * _K
        base = r0 * 128
        pltpu.sync_copy(idx_hbm.at[pl.ds(r0, _K)], idx_v)
        copies = [
            pltpu.async_copy(tab_hbm.at[idx_v.at[j]],
                             pe_v.at[pl.ds(j * 128, 128)], gsem)
            for j in range(_K)
        ]
        emb_cp = pltpu.async_copy(
            emb_hbm.at[pl.ds(base, _C)],
            out_hbm.at[pl.ds(base, _C), pl.ds(0, _D)], esem)
        for c in copies:
            c.wait()
        pltpu.sync_copy(pe_v, out_hbm.at[pl.ds(base, _C), pl.ds(_D, _D)])
        emb_cp.wait()
        return carry

    lax.fori_loop(0, _CHUNKS, chunk, 0)


def kernel(input, embedded, pos_emb):
    idx = input.reshape(_N // 128, 128).astype(jnp.int32)
    emb = embedded.reshape(_N, _D)
    mesh = plsc.VectorSubcoreMesh(core_axis_name="c", subcore_axis_name="s")
    out = pl.kernel(
        _sc_body,
        out_type=jax.ShapeDtypeStruct((_N, 2 * _D), jnp.float32),
        mesh=mesh,
        scratch_types=[
            pltpu.VMEM((_K, 128), jnp.int32),
            pltpu.VMEM((_C, _D), jnp.float32),
            pltpu.SemaphoreType.DMA,
            pltpu.SemaphoreType.DMA,
        ],
        compiler_params=pltpu.CompilerParams(use_tc_tiling_on_sc=False),
    )(idx, emb, pos_emb)
    return out.reshape(_B, _L, 2 * _D)


# speedup vs baseline: 8.4452x; 8.4452x over previous
import jax
import jax.numpy as jnp
from jax import lax
from jax.experimental import pallas as pl
from jax.experimental.pallas import tpu as pltpu
from jax.experimental.pallas import tpu_sc as plsc

_B, _L, _D = 4096, 200, 64
_N = _B * _L
_NC, _NS = 2, 16
_NW = _NC * _NS
_K = 2
_C = _K * 128
_CHUNKS = _N // (_NW * _C)


def _sc_body(idx_hbm, emb_hbm, tab_hbm, out_hbm, *s):
    idx_v = s[0:2]
    pe_v = s[2:4]
    emb_v = s[4:6]
    isem = s[6:8]
    esem = s[8:10]
    gsem = s[10:12]
    wsem = s[12:14]
    psem = s[14:16]
    wid = lax.axis_index("s") * _NC + lax.axis_index("c")
    wbase = wid * _CHUNKS

    def issue_reads(c, p):
        r0 = (wbase + c) * _K
        pltpu.async_copy(idx_hbm.at[pl.ds(r0, _K)], idx_v[p], isem[p])
        pltpu.async_copy(emb_hbm.at[pl.ds(r0 * 128, _C)], emb_v[p], esem[p])

    def wait_reads(p):
        pltpu.make_async_copy(idx_hbm.at[pl.ds(0, _K)], idx_v[p], isem[p]).wait()
        pltpu.make_async_copy(emb_hbm.at[pl.ds(0, _C)], emb_v[p], esem[p]).wait()

    def wait_writes(p):
        pltpu.make_async_copy(
            emb_v[p], out_hbm.at[pl.ds(0, _C), pl.ds(0, _D)], wsem[p]).wait()
        pltpu.make_async_copy(
            pe_v[p], out_hbm.at[pl.ds(0, _C), pl.ds(_D, _D)], psem[p]).wait()

    def fire_gathers(p):
        for j in range(_K):
            pltpu.async_copy(tab_hbm.at[idx_v[p].at[j]],
                             pe_v[p].at[pl.ds(j * 128, 128)], gsem[p])

    def wait_gathers(p):
        pltpu.make_async_copy(tab_hbm.at[pl.ds(0, _C)], pe_v[p], gsem[p]).wait()

    def issue_writes(c, p):
        base = (wbase + c) * _C
        pltpu.async_copy(emb_v[p],
                         out_hbm.at[pl.ds(base, _C), pl.ds(0, _D)], wsem[p])

    def issue_pe_write(c, p):
        base = (wbase + c) * _C
        pltpu.async_copy(pe_v[p],
                         out_hbm.at[pl.ds(base, _C), pl.ds(_D, _D)], psem[p])

    def step(c, p, first):
        q = 1 - p
        if not first:
            wait_writes(q)
        issue_reads(jnp.minimum(c + 1, _CHUNKS - 1), q)
        wait_reads(p)
        fire_gathers(p)
        issue_writes(c, p)
        wait_gathers(p)
        issue_pe_write(c, p)

    issue_reads(0, 0)
    step(0, 0, True)
    step(1, 1, False)

    def loop(k, carry):
        step(2 * k, 0, False)
        step(2 * k + 1, 1, False)
        return carry

    lax.fori_loop(1, _CHUNKS // 2, loop, 0)

    wait_writes(1)
    wait_reads(0)


def kernel(input, embedded, pos_emb):
    idx = input.reshape(_N // 128, 128).astype(jnp.int32)
    emb = embedded.reshape(_N, _D)
    mesh = plsc.VectorSubcoreMesh(core_axis_name="c", subcore_axis_name="s")
    out = pl.kernel(
        _sc_body,
        out_type=jax.ShapeDtypeStruct((_N, 2 * _D), jnp.float32),
        mesh=mesh,
        scratch_types=(
            [pltpu.VMEM((_K, 128), jnp.int32) for _ in range(2)]
            + [pltpu.VMEM((_C, _D), jnp.float32) for _ in range(4)]
            + [pltpu.SemaphoreType.DMA for _ in range(10)]
        ),
        compiler_params=pltpu.CompilerParams(use_tc_tiling_on_sc=False),
    )(idx, emb, pos_emb)
    return out.reshape(_B, _L, 2 * _D)


# speedup vs baseline: 9.3732x; 1.1099x over previous
import jax
import jax.numpy as jnp
from jax import lax
from jax.experimental import pallas as pl
from jax.experimental.pallas import tpu as pltpu
from jax.experimental.pallas import tpu_sc as plsc

_B, _L, _D = 4096, 200, 64
_N = _B * _L
_NC, _NS = 2, 16
_NW = _NC * _NS
_K = 2
_C = _K * 128
_CHUNKS = _N // (_NW * _C)


def _sc_body(idx_hbm, emb_hbm, tab_hbm, out_hbm, *s):
    idx_v = s[0:2]
    pe_v = s[2:4]
    emb_v = s[4:6]
    isem = s[6:8]
    esem = s[8:10]
    gsem = s[10:12]
    wsem = s[12:14]
    psem = s[14:16]
    tab_sh = s[16]
    wid = lax.axis_index("s") * _NC + lax.axis_index("c")
    wbase = wid * _CHUNKS

    @pl.when(lax.axis_index("s") == 0)
    def _():
        pltpu.sync_copy(tab_hbm, tab_sh)
    plsc.subcore_barrier()

    def issue_reads(c, p):
        r0 = (wbase + c) * _K
        pltpu.async_copy(idx_hbm.at[pl.ds(r0, _K)], idx_v[p], isem[p])
        pltpu.async_copy(emb_hbm.at[pl.ds(r0 * 128, _C)], emb_v[p], esem[p])

    def wait_reads(p):
        pltpu.make_async_copy(idx_hbm.at[pl.ds(0, _K)], idx_v[p], isem[p]).wait()
        pltpu.make_async_copy(emb_hbm.at[pl.ds(0, _C)], emb_v[p], esem[p]).wait()

    def wait_writes(p):
        pltpu.make_async_copy(
            emb_v[p], out_hbm.at[pl.ds(0, _C), pl.ds(0, _D)], wsem[p]).wait()
        pltpu.make_async_copy(
            pe_v[p], out_hbm.at[pl.ds(0, _C), pl.ds(_D, _D)], psem[p]).wait()

    def fire_gathers(p):
        for j in range(_K):
            pltpu.async_copy(tab_sh.at[idx_v[p].at[j]],
                             pe_v[p].at[pl.ds(j * 128, 128)], gsem[p])

    def wait_gathers(p):
        pltpu.make_async_copy(tab_hbm.at[pl.ds(0, _C)], pe_v[p], gsem[p]).wait()

    def issue_writes(c, p):
        base = (wbase + c) * _C
        pltpu.async_copy(emb_v[p],
                         out_hbm.at[pl.ds(base, _C), pl.ds(0, _D)], wsem[p])

    def issue_pe_write(c, p):
        base = (wbase + c) * _C
        pltpu.async_copy(pe_v[p],
                         out_hbm.at[pl.ds(base, _C), pl.ds(_D, _D)], psem[p])

    def step(c, p, first):
        q = 1 - p
        if not first:
            wait_writes(q)
        issue_reads(jnp.minimum(c + 1, _CHUNKS - 1), q)
        wait_reads(p)
        fire_gathers(p)
        issue_writes(c, p)
        wait_gathers(p)
        issue_pe_write(c, p)

    issue_reads(0, 0)
    step(0, 0, True)
    step(1, 1, False)

    def loop(k, carry):
        step(2 * k, 0, False)
        step(2 * k + 1, 1, False)
        return carry

    lax.fori_loop(1, _CHUNKS // 2, loop, 0)

    wait_writes(1)
    wait_reads(0)


def kernel(input, embedded, pos_emb):
    idx = input.reshape(_N // 128, 128).astype(jnp.int32)
    emb = embedded.reshape(_N, _D)
    mesh = plsc.VectorSubcoreMesh(core_axis_name="c", subcore_axis_name="s")
    out = pl.kernel(
        _sc_body,
        out_type=jax.ShapeDtypeStruct((_N, 2 * _D), jnp.float32),
        mesh=mesh,
        scratch_types=(
            [pltpu.VMEM((_K, 128), jnp.int32) for _ in range(2)]
            + [pltpu.VMEM((_C, _D), jnp.float32) for _ in range(4)]
            + [pltpu.SemaphoreType.DMA for _ in range(10)]
            + [pltpu.VMEM_SHARED((4096, _D), jnp.float32)]
        ),
        compiler_params=pltpu.CompilerParams(use_tc_tiling_on_sc=False),
    )(idx, emb, pos_emb)
    return out.reshape(_B, _L, 2 * _D)


# speedup vs baseline: 9.3844x; 1.0012x over previous
import jax
import jax.numpy as jnp
from jax import lax
from jax.experimental import pallas as pl
from jax.experimental.pallas import tpu as pltpu
from jax.experimental.pallas import tpu_sc as plsc

_B, _L, _D = 4096, 200, 64
_N = _B * _L
_NC, _NS = 2, 16
_NW = _NC * _NS
_K = 2
_C = _K * 128
_CHUNKS = _N // (_NW * _C)
_NBUF = 3


def _sc_body(idx_hbm, emb_hbm, tab_hbm, out_hbm, *s):
    idx_v = s[0:3]
    pe_v = s[3:6]
    emb_v = s[6:9]
    isem = s[9:12]
    esem = s[12:15]
    gsem = s[15:18]
    wsem = s[18:21]
    psem = s[21:24]
    tab_sh = s[24]
    wid = lax.axis_index("s") * _NC + lax.axis_index("c")
    wbase = wid * _CHUNKS

    @pl.when(lax.axis_index("s") == 0)
    def _():
        pltpu.sync_copy(tab_hbm, tab_sh)
    plsc.subcore_barrier()

    def issue_reads(c, b):
        r0 = (wbase + c) * _K
        pltpu.async_copy(idx_hbm.at[pl.ds(r0, _K)], idx_v[b], isem[b])
        pltpu.async_copy(emb_hbm.at[pl.ds(r0 * 128, _C)], emb_v[b], esem[b])

    def wait_reads(b):
        pltpu.make_async_copy(idx_hbm.at[pl.ds(0, _K)], idx_v[b], isem[b]).wait()
        pltpu.make_async_copy(emb_hbm.at[pl.ds(0, _C)], emb_v[b], esem[b]).wait()

    def wait_writes(b):
        pltpu.make_async_copy(
            emb_v[b], out_hbm.at[pl.ds(0, _C), pl.ds(0, _D)], wsem[b]).wait()
        pltpu.make_async_copy(
            pe_v[b], out_hbm.at[pl.ds(0, _C), pl.ds(_D, _D)], psem[b]).wait()

    def step(c, b, first=False, last=False):
        wait_reads(b)
        for j in range(_K):
            pltpu.async_copy(tab_sh.at[idx_v[b].at[j]],
                             pe_v[b].at[pl.ds(j * 128, 128)], gsem[b])
        base = (wbase + c) * _C
        pltpu.async_copy(emb_v[b],
                         out_hbm.at[pl.ds(base, _C), pl.ds(0, _D)], wsem[b])
        pltpu.make_async_copy(tab_hbm.at[pl.ds(0, _C)], pe_v[b], gsem[b]).wait()
        pltpu.async_copy(pe_v[b],
                         out_hbm.at[pl.ds(base, _C), pl.ds(_D, _D)], psem[b])
        bn = (b + 2) % _NBUF
        if not first:
            wait_writes(bn)
        if not last:
            issue_reads(jnp.minimum(c + 2, _CHUNKS - 1), bn)

    issue_reads(0, 0)
    issue_reads(1, 1)
    step(0, 0, first=True)
    step(1, 1)
    step(2, 2)

    def loop(k, carry):
        step(3 * k, 0)
        step(3 * k + 1, 1)
        step(3 * k + 2, 2)
        return carry

    lax.fori_loop(1, _CHUNKS // 3, loop, 0)
    step(_CHUNKS - 1, (_CHUNKS - 1) % _NBUF, last=True)

    wait_writes((_CHUNKS - 1) % _NBUF)
    wait_reads(_CHUNKS % _NBUF)


def kernel(input, embedded, pos_emb):
    idx = input.reshape(_N // 128, 128).astype(jnp.int32)
    emb = embedded.reshape(_N, _D)
    mesh = plsc.VectorSubcoreMesh(core_axis_name="c", subcore_axis_name="s")
    out = pl.kernel(
        _sc_body,
        out_type=jax.ShapeDtypeStruct((_N, 2 * _D), jnp.float32),
        mesh=mesh,
        scratch_types=(
            [pltpu.VMEM((_K, 128), jnp.int32) for _ in range(_NBUF)]
            + [pltpu.VMEM((_C, _D), jnp.float32) for _ in range(2 * _NBUF)]
            + [pltpu.SemaphoreType.DMA for _ in range(5 * _NBUF)]
            + [pltpu.VMEM_SHARED((4096, _D), jnp.float32)]
        ),
        compiler_params=pltpu.CompilerParams(use_tc_tiling_on_sc=False),
    )(idx, emb, pos_emb)
    return out.reshape(_B, _L, 2 * _D)
